# baseline (speedup 1.0000x reference)
import jax
import jax.numpy as jnp
from jax import lax
from jax.experimental import pallas as pl

_TN = 256


def _pass1_body(cnt, unknown_ref, known_ref, kfeat_ref, ufeat_ref, w1_ref,
                hpre_ref, stat_ref):
    del cnt
    TN = unknown_ref.shape[1]
    M = known_ref.shape[2]
    u = unknown_ref[0]
    k = known_ref[0]
    uk = lax.dot_general(u, k[:3], (((1,), (0,)), ((), ())),
                         preferred_element_type=jnp.float32,
                         precision=lax.Precision.HIGHEST)
    unorm = jnp.sum(u * u, axis=1, keepdims=True)
    knorm = k[0:1] ** 2 + k[1:2] ** 2 + k[2:3] ** 2
    d2 = (unorm + knorm) - 2.0 * uk

    iota = lax.broadcasted_iota(jnp.int32, (TN, M), 1).astype(jnp.float32)
    inf = jnp.float32(jnp.inf)
    fM = jnp.float32(M)

    v1 = jnp.min(d2, axis=1, keepdims=True)
    i1 = jnp.min(jnp.where(d2 == v1, iota, fM), axis=1, keepdims=True)
    d2b = jnp.where(iota == i1, inf, d2)
    v2 = jnp.min(d2b, axis=1, keepdims=True)
    i2 = jnp.min(jnp.where(d2b == v2, iota, fM), axis=1, keepdims=True)
    d2c = jnp.where(iota == i2, inf, d2b)
    v3 = jnp.min(d2c, axis=1, keepdims=True)
    i3 = jnp.min(jnp.where(d2c == v3, iota, fM), axis=1, keepdims=True)

    r1 = 1.0 / (jnp.sqrt(jnp.maximum(v1, 0.0)) + 1e-8)
    r2 = 1.0 / (jnp.sqrt(jnp.maximum(v2, 0.0)) + 1e-8)
    r3 = 1.0 / (jnp.sqrt(jnp.maximum(v3, 0.0)) + 1e-8)
    norm = r1 + r2 + r3
    zero = jnp.float32(0.0)
    oh = (jnp.where(iota == i1, r1 / norm, zero)
          + jnp.where(iota == i2, r2 / norm, zero)
          + jnp.where(iota == i3, r3 / norm, zero))

    kf = kfeat_ref[0]
    interp = lax.dot_general(kf, oh, (((1,), (1,)), ((), ())),
                             preferred_element_type=jnp.float32)
    uf = ufeat_ref[0]
    W1 = w1_ref[...]
    C2 = kf.shape[0]
    h = (lax.dot_general(W1[:, :C2], interp, (((1,), (0,)), ((), ())),
                         preferred_element_type=jnp.float32)
         + lax.dot_general(W1[:, C2:], uf, (((1,), (0,)), ((), ())),
                           preferred_element_type=jnp.float32))
    hpre_ref[0] = h

    first = (pl.program_id(0) == 0) & (pl.program_id(1) == 0)

    @pl.when(first)
    def _():
        stat_ref[...] = jnp.zeros_like(stat_ref)

    stat_ref[:, 0:1] += jnp.sum(h, axis=1, keepdims=True)
    stat_ref[:, 1:2] += jnp.sum(h * h, axis=1, keepdims=True)


def _pass2_body(cnt, hpre_ref, stat_ref, gamma_ref, beta_ref, wr_ref, br_ref,
                we_ref, be_ref, out_ref):
    h = hpre_ref[0]
    mean = stat_ref[:, 0:1] / cnt
    var = stat_ref[:, 1:2] / cnt - mean * mean
    hn = (h - mean) / jnp.sqrt(var + 1e-5) * gamma_ref[...] + beta_ref[...]
    hn = jnp.maximum(hn, 0.0)
    s = lax.dot_general(wr_ref[...], hn, (((1,), (0,)), ((), ())),
                        preferred_element_type=jnp.float32) + br_ref[...]
    s = s * jax.nn.sigmoid(s)
    e = lax.dot_general(we_ref[...], s, (((1,), (0,)), ((), ())),
                        preferred_element_type=jnp.float32) + be_ref[...]
    out_ref[0] = jax.nn.sigmoid(e) * hn


def kernel(unknown, known, unknow_feats, known_feats, W1, gamma, beta, Wr, br,
           We, be):
    import functools

    B, N, _ = unknown.shape
    M = known.shape[1]
    C2 = known_feats.shape[1]
    C1 = unknow_feats.shape[1]
    COUT, CIN = W1.shape
    NSQ = Wr.shape[0]
    TN = _TN
    NT = N // TN
    cnt = float(B * N)

    known_t = jnp.pad(jnp.transpose(known, (0, 2, 1)),
                      ((0, 0), (0, 5), (0, 0)))

    hpre, stat = pl.pallas_call(
        functools.partial(_pass1_body, cnt),
        grid=(B, NT),
        in_specs=[
            pl.BlockSpec((1, TN, 3), lambda b, t: (b, t, 0)),
            pl.BlockSpec((1, 8, M), lambda b, t: (b, 0, 0)),
            pl.BlockSpec((1, C2, M), lambda b, t: (b, 0, 0)),
            pl.BlockSpec((1, C1, TN), lambda b, t: (b, 0, t)),
            pl.BlockSpec((COUT, CIN), lambda b, t: (0, 0)),
        ],
        out_specs=[
            pl.BlockSpec((1, COUT, TN), lambda b, t: (b, 0, t)),
            pl.BlockSpec((COUT, 2), lambda b, t: (0, 0)),
        ],
        out_shape=[
            jax.ShapeDtypeStruct((B, COUT, N), jnp.float32),
            jax.ShapeDtypeStruct((COUT, 2), jnp.float32),
        ],
    )(unknown, known_t, known_feats, unknow_feats, W1)

    NSQP = 8
    wr_p = jnp.pad(Wr, ((0, NSQP - NSQ), (0, 0)))
    br_p = jnp.pad(br, (0, NSQP - NSQ)).reshape(NSQP, 1)
    we_p = jnp.pad(We, ((0, 0), (0, NSQP - NSQ)))
    gamma_c = gamma.reshape(COUT, 1)
    beta_c = beta.reshape(COUT, 1)
    be_c = be.reshape(COUT, 1)

    out = pl.pallas_call(
        functools.partial(_pass2_body, cnt),
        grid=(B, NT),
        in_specs=[
            pl.BlockSpec((1, COUT, TN), lambda b, t: (b, 0, t)),
            pl.BlockSpec((COUT, 2), lambda b, t: (0, 0)),
            pl.BlockSpec((COUT, 1), lambda b, t: (0, 0)),
            pl.BlockSpec((COUT, 1), lambda b, t: (0, 0)),
            pl.BlockSpec((NSQP, COUT), lambda b, t: (0, 0)),
            pl.BlockSpec((NSQP, 1), lambda b, t: (0, 0)),
            pl.BlockSpec((COUT, NSQP), lambda b, t: (0, 0)),
            pl.BlockSpec((COUT, 1), lambda b, t: (0, 0)),
        ],
        out_specs=pl.BlockSpec((1, COUT, TN), lambda b, t: (b, 0, t)),
        out_shape=jax.ShapeDtypeStruct((B, COUT, N), jnp.float32),
    )(hpre, stat, gamma_c, beta_c, wr_p, br_p, we_p, be_c)
    return out


# speedup vs baseline: 1.3185x; 1.3185x over previous
import jax
import jax.numpy as jnp
from jax import lax
from jax.experimental import pallas as pl

_TN = 256


def _pass1_body(cnt, unknown_ref, known_ref, kfeat_ref, ufeat_ref, w1_ref,
                hpre_ref, stat_ref):
    del cnt
    TN = unknown_ref.shape[1]
    M = known_ref.shape[2]
    u = unknown_ref[0]
    k = known_ref[0]
    d2 = ((u[:, 0:1] - k[0:1, :]) ** 2
          + (u[:, 1:2] - k[1:2, :]) ** 2
          + (u[:, 2:3] - k[2:3, :]) ** 2)

    iota = lax.broadcasted_iota(jnp.int32, (TN, M), 1).astype(jnp.float32)
    inf = jnp.float32(jnp.inf)
    fM = jnp.float32(M)

    v1 = jnp.min(d2, axis=1, keepdims=True)
    i1 = jnp.min(jnp.where(d2 == v1, iota, fM), axis=1, keepdims=True)
    d2b = jnp.where(iota == i1, inf, d2)
    v2 = jnp.min(d2b, axis=1, keepdims=True)
    i2 = jnp.min(jnp.where(d2b == v2, iota, fM), axis=1, keepdims=True)
    d2c = jnp.where(iota == i2, inf, d2b)
    v3 = jnp.min(d2c, axis=1, keepdims=True)
    i3 = jnp.min(jnp.where(d2c == v3, iota, fM), axis=1, keepdims=True)

    r1 = 1.0 / (jnp.sqrt(jnp.maximum(v1, 0.0)) + 1e-8)
    r2 = 1.0 / (jnp.sqrt(jnp.maximum(v2, 0.0)) + 1e-8)
    r3 = 1.0 / (jnp.sqrt(jnp.maximum(v3, 0.0)) + 1e-8)
    norm = r1 + r2 + r3
    zero = jnp.float32(0.0)
    oh = (jnp.where(iota == i1, r1 / norm, zero)
          + jnp.where(iota == i2, r2 / norm, zero)
          + jnp.where(iota == i3, r3 / norm, zero))

    kf = kfeat_ref[0]
    interp = lax.dot_general(kf, oh, (((1,), (1,)), ((), ())),
                             preferred_element_type=jnp.float32)
    uf = ufeat_ref[0]
    W1 = w1_ref[...]
    C2 = kf.shape[0]
    h = (lax.dot_general(W1[:, :C2], interp, (((1,), (0,)), ((), ())),
                         preferred_element_type=jnp.float32)
         + lax.dot_general(W1[:, C2:], uf, (((1,), (0,)), ((), ())),
                           preferred_element_type=jnp.float32))
    hpre_ref[0] = h

    first = (pl.program_id(0) == 0) & (pl.program_id(1) == 0)

    @pl.when(first)
    def _():
        stat_ref[...] = jnp.zeros_like(stat_ref)

    stat_ref[:, 0:1] += jnp.sum(h, axis=1, keepdims=True)
    stat_ref[:, 1:2] += jnp.sum(h * h, axis=1, keepdims=True)


def _pass2_body(cnt, hpre_ref, stat_ref, gamma_ref, beta_ref, wr_ref, br_ref,
                we_ref, be_ref, out_ref):
    h = hpre_ref[0]
    mean = stat_ref[:, 0:1] / cnt
    var = stat_ref[:, 1:2] / cnt - mean * mean
    hn = (h - mean) / jnp.sqrt(var + 1e-5) * gamma_ref[...] + beta_ref[...]
    hn = jnp.maximum(hn, 0.0)
    s = lax.dot_general(wr_ref[...], hn, (((1,), (0,)), ((), ())),
                        preferred_element_type=jnp.float32) + br_ref[...]
    s = s * jax.nn.sigmoid(s)
    e = lax.dot_general(we_ref[...], s, (((1,), (0,)), ((), ())),
                        preferred_element_type=jnp.float32) + be_ref[...]
    out_ref[0] = jax.nn.sigmoid(e) * hn


def kernel(unknown, known, unknow_feats, known_feats, W1, gamma, beta, Wr, br,
           We, be):
    import functools

    B, N, _ = unknown.shape
    M = known.shape[1]
    C2 = known_feats.shape[1]
    C1 = unknow_feats.shape[1]
    COUT, CIN = W1.shape
    NSQ = Wr.shape[0]
    TN = _TN
    NT = N // TN
    cnt = float(B * N)

    known_t = jnp.pad(jnp.transpose(known, (0, 2, 1)),
                      ((0, 0), (0, 5), (0, 0)))

    hpre, stat = pl.pallas_call(
        functools.partial(_pass1_body, cnt),
        grid=(B, NT),
        in_specs=[
            pl.BlockSpec((1, TN, 3), lambda b, t: (b, t, 0)),
            pl.BlockSpec((1, 8, M), lambda b, t: (b, 0, 0)),
            pl.BlockSpec((1, C2, M), lambda b, t: (b, 0, 0)),
            pl.BlockSpec((1, C1, TN), lambda b, t: (b, 0, t)),
            pl.BlockSpec((COUT, CIN), lambda b, t: (0, 0)),
        ],
        out_specs=[
            pl.BlockSpec((1, COUT, TN), lambda b, t: (b, 0, t)),
            pl.BlockSpec((COUT, 2), lambda b, t: (0, 0)),
        ],
        out_shape=[
            jax.ShapeDtypeStruct((B, COUT, N), jnp.float32),
            jax.ShapeDtypeStruct((COUT, 2), jnp.float32),
        ],
    )(unknown, known_t, known_feats, unknow_feats, W1)

    NSQP = 8
    wr_p = jnp.pad(Wr, ((0, NSQP - NSQ), (0, 0)))
    br_p = jnp.pad(br, (0, NSQP - NSQ)).reshape(NSQP, 1)
    we_p = jnp.pad(We, ((0, 0), (0, NSQP - NSQ)))
    gamma_c = gamma.reshape(COUT, 1)
    beta_c = beta.reshape(COUT, 1)
    be_c = be.reshape(COUT, 1)

    out = pl.pallas_call(
        functools.partial(_pass2_body, cnt),
        grid=(B, NT),
        in_specs=[
            pl.BlockSpec((1, COUT, TN), lambda b, t: (b, 0, t)),
            pl.BlockSpec((COUT, 2), lambda b, t: (0, 0)),
            pl.BlockSpec((COUT, 1), lambda b, t: (0, 0)),
            pl.BlockSpec((COUT, 1), lambda b, t: (0, 0)),
            pl.BlockSpec((NSQP, COUT), lambda b, t: (0, 0)),
            pl.BlockSpec((NSQP, 1), lambda b, t: (0, 0)),
            pl.BlockSpec((COUT, NSQP), lambda b, t: (0, 0)),
            pl.BlockSpec((COUT, 1), lambda b, t: (0, 0)),
        ],
        out_specs=pl.BlockSpec((1, COUT, TN), lambda b, t: (b, 0, t)),
        out_shape=jax.ShapeDtypeStruct((B, COUT, N), jnp.float32),
    )(hpre, stat, gamma_c, beta_c, wr_p, br_p, we_p, be_c)
    return out
